# Initial kernel scaffold; baseline (speedup 1.0000x reference)
import jax
import jax.numpy as jnp
from jax.experimental import pallas as pl


def kernel(query, key, value, Wq, bq, Wk, bk, Wv, bv, Wo, bo, Wc1, bc1, Wc2, bc2, path_mixer):
    raise NotImplementedError("write your pallas kernel here")


# speedup vs baseline: 2.1592x; 2.1592x over previous
import functools
import math

import jax
import jax.numpy as jnp
from jax import lax
from jax.experimental import pallas as pl

L = 2048
D = 768
H = 12
DH = 64
LC = 512
CR = 4
HALF = 64
U = 8
TQ = 256
SCALE = 1.0 / math.sqrt(DH)
LN_L = math.log(L)


def _dotT(a, b):
    return lax.dot_general(a, b, (((1,), (1,)), ((), ())))


def _dot(a, b):
    return lax.dot_general(a, b, (((1,), (0,)), ((), ())))


def _split_heads(x):
    return x.reshape(x.shape[0], H, DH).transpose(1, 0, 2)


def _proj_kernel(xq_ref, xk_ref, xv_ref, wq_ref, bq_ref, wk_ref, bk_ref,
                 wv_ref, bv_ref, wc1_ref, bc1_ref, wc2_ref, bc2_ref,
                 q_ref, k_ref, v_ref, kc_ref, vc_ref):
    q_t = _dotT(xq_ref[...], wq_ref[...]) + bq_ref[...]
    k_t = _dotT(xk_ref[...], wk_ref[...]) + bk_ref[...]
    v_t = _dotT(xv_ref[...], wv_ref[...]) + bv_ref[...]
    q_ref[...] = _split_heads(q_t)
    k_ref[...] = _split_heads(k_t)
    v_ref[...] = _split_heads(v_t)
    rows = lax.broadcasted_iota(jnp.int32, (TQ // CR, TQ), 0)
    cols = lax.broadcasted_iota(jnp.int32, (TQ // CR, TQ), 1)
    pool = jnp.where((cols >= rows * CR) & (cols < rows * CR + CR),
                     1.0 / CR, 0.0).astype(jnp.float32)
    k_c = _dot(pool, k_t)
    vc_ref[...] = _split_heads(_dot(pool, v_t))
    h1 = _dotT(k_c, wc1_ref[...]) + bc1_ref[...]
    g = 0.5 * h1 * (1.0 + lax.erf(h1 / math.sqrt(2.0)))
    kc_ref[...] = _split_heads(_dotT(g, wc2_ref[...]) + bc2_ref[...])


def _attn_kernel(pm_ref, q_ref, k_ref, v_ref, kg_ref, vg_ref,
                 awg_ref, awl_ref, comb_ref, imp_ref):
    qi = pl.program_id(1)
    pm = pm_ref[...]
    e = jnp.exp(pm - jnp.max(pm))
    pw = e / jnp.sum(e)
    pw0 = pw[0, 0]
    pw1 = pw[0, 1]

    q = q_ref[0]

    sg = _dotT(q, kg_ref[0]) * SCALE
    pg = jnp.exp(sg - jnp.max(sg, axis=1, keepdims=True))
    awg = pg / jnp.sum(pg, axis=1, keepdims=True)
    awg_ref[0, 0] = awg
    g_out = _dot(awg, vg_ref[0])

    s = _dotT(q, k_ref[0]) * SCALE
    ms = jnp.max(s, axis=1, keepdims=True)
    p = jnp.exp(s - ms)
    sum_p = jnp.sum(p, axis=1, keepdims=True)
    lse = jnp.log(sum_p) + ms
    mean_s = jnp.sum(s, axis=1, keepdims=True) * (1.0 / L)
    imp = (lse - LN_L) - mean_s
    imp_ref[0, 0, pl.ds(qi * TQ, TQ)] = imp[:, 0]

    rows = qi * TQ + lax.broadcasted_iota(jnp.int32, (TQ, L), 0)
    cols = lax.broadcasted_iota(jnp.int32, (TQ, L), 1)
    band = jnp.abs(rows - cols) <= HALF
    pb = jnp.where(band, p, 0.0)
    awl = pb / jnp.sum(pb, axis=1, keepdims=True)
    awl_ref[0, 0] = awl
    l_out = _dot(awl, v_ref[0])

    comb_ref[0] = pw0 * g_out + pw1 * l_out


def _topk_kernel(imp_ref, top_ref):
    x = imp_ref[:, 0, :]
    idx = lax.broadcasted_iota(jnp.int32, (H, L), 1)
    cols = []
    for _ in range(U):
        m = jnp.max(x, axis=1, keepdims=True)
        cand = jnp.where(x == m, idx, L)
        am = jnp.min(cand, axis=1)
        cols.append(am.reshape(H, 1))
        x = jnp.where(idx == am[:, None], -jnp.inf, x)
    top_ref[...] = jnp.concatenate(cols, axis=1)


def _sel_kernel(top_ref, q_ref, k_ref, v_ref, sel_ref):
    h = pl.program_id(0)
    t = top_ref[pl.ds(h, 1), :]
    colid = lax.broadcasted_iota(jnp.int32, (L, U), 0)
    onehot = (colid == t).astype(jnp.float32)
    q_sel = lax.dot_general(onehot, q_ref[0],
                            (((0,), (0,)), ((), ())))
    s = _dotT(q_sel, k_ref[0]) * SCALE
    p = jnp.exp(s - jnp.max(s, axis=1, keepdims=True))
    aw = p / jnp.sum(p, axis=1, keepdims=True)
    sel_ref[0] = _dot(aw, v_ref[0])


def _out_kernel(pm_ref, top_ref, sel_ref, comb_ref, wo_ref, bo_ref, out_ref):
    li = pl.program_id(0)
    pm = pm_ref[...]
    e = jnp.exp(pm - jnp.max(pm))
    pw = e / jnp.sum(e)
    pw2 = pw[0, 2]
    rows = li * TQ + lax.broadcasted_iota(jnp.int32, (TQ, U), 0)
    adds = []
    for h in range(H):
        th = top_ref[h:h + 1, :]
        oh = (rows == th).astype(jnp.float32)
        adds.append(_dot(oh, sel_ref[h]))
    addm = jnp.concatenate(adds, axis=1)
    comb = comb_ref[...]
    x = comb.transpose(1, 0, 2).reshape(TQ, D) + pw2 * addm
    out_ref[...] = _dotT(x, wo_ref[...]) + bo_ref[...]


def kernel(query, key, value, Wq, bq, Wk, bk, Wv, bv, Wo, bo,
           Wc1, bc1, Wc2, bc2, path_mixer):
    f32 = jnp.float32
    xq = query.reshape(L, D)
    xk = key.reshape(L, D)
    xv = value.reshape(L, D)
    b2 = lambda b: b.reshape(1, D)
    pm = path_mixer.reshape(1, 3)

    wspec = pl.BlockSpec((D, D), lambda *_: (0, 0))
    bspec = pl.BlockSpec((1, D), lambda *_: (0, 0))
    nlt = L // TQ

    q, k, v, kc, vc = pl.pallas_call(
        _proj_kernel,
        grid=(nlt,),
        in_specs=[
            pl.BlockSpec((TQ, D), lambda i: (i, 0)),
            pl.BlockSpec((TQ, D), lambda i: (i, 0)),
            pl.BlockSpec((TQ, D), lambda i: (i, 0)),
            wspec, bspec, wspec, bspec, wspec, bspec,
            wspec, bspec, wspec, bspec,
        ],
        out_specs=[
            pl.BlockSpec((H, TQ, DH), lambda i: (0, i, 0)),
            pl.BlockSpec((H, TQ, DH), lambda i: (0, i, 0)),
            pl.BlockSpec((H, TQ, DH), lambda i: (0, i, 0)),
            pl.BlockSpec((H, TQ // CR, DH), lambda i: (0, i, 0)),
            pl.BlockSpec((H, TQ // CR, DH), lambda i: (0, i, 0)),
        ],
        out_shape=[
            jax.ShapeDtypeStruct((H, L, DH), f32),
            jax.ShapeDtypeStruct((H, L, DH), f32),
            jax.ShapeDtypeStruct((H, L, DH), f32),
            jax.ShapeDtypeStruct((H, LC, DH), f32),
            jax.ShapeDtypeStruct((H, LC, DH), f32),
        ],
    )(xq, xk, xv, Wq, b2(bq), Wk, b2(bk), Wv, b2(bv),
      Wc1, b2(bc1), Wc2, b2(bc2))

    awg, awl, comb, imp = pl.pallas_call(
        _attn_kernel,
        grid=(H, nlt),
        in_specs=[
            pl.BlockSpec((1, 3), lambda h, i: (0, 0)),
            pl.BlockSpec((1, TQ, DH), lambda h, i: (h, i, 0)),
            pl.BlockSpec((1, L, DH), lambda h, i: (h, 0, 0)),
            pl.BlockSpec((1, L, DH), lambda h, i: (h, 0, 0)),
            pl.BlockSpec((1, LC, DH), lambda h, i: (h, 0, 0)),
            pl.BlockSpec((1, LC, DH), lambda h, i: (h, 0, 0)),
        ],
        out_specs=[
            pl.BlockSpec((1, 1, TQ, LC), lambda h, i: (0, h, i, 0)),
            pl.BlockSpec((1, 1, TQ, L), lambda h, i: (0, h, i, 0)),
            pl.BlockSpec((1, TQ, DH), lambda h, i: (h, i, 0)),
            pl.BlockSpec((1, 1, L), lambda h, i: (h, 0, 0)),
        ],
        out_shape=[
            jax.ShapeDtypeStruct((1, H, L, LC), f32),
            jax.ShapeDtypeStruct((1, H, L, L), f32),
            jax.ShapeDtypeStruct((H, L, DH), f32),
            jax.ShapeDtypeStruct((H, 1, L), f32),
        ],
    )(pm, q, k, v, kc, vc)

    top = pl.pallas_call(
        _topk_kernel,
        grid=(1,),
        in_specs=[pl.BlockSpec((H, 1, L), lambda i: (0, 0, 0))],
        out_specs=pl.BlockSpec((H, U), lambda i: (0, 0)),
        out_shape=jax.ShapeDtypeStruct((H, U), jnp.int32),
    )(imp)

    sel = pl.pallas_call(
        _sel_kernel,
        grid=(H,),
        in_specs=[
            pl.BlockSpec((H, U), lambda h: (0, 0)),
            pl.BlockSpec((1, L, DH), lambda h: (h, 0, 0)),
            pl.BlockSpec((1, L, DH), lambda h: (h, 0, 0)),
            pl.BlockSpec((1, L, DH), lambda h: (h, 0, 0)),
        ],
        out_specs=pl.BlockSpec((1, U, DH), lambda h: (h, 0, 0)),
        out_shape=jax.ShapeDtypeStruct((H, U, DH), f32),
    )(top, q, k, v)

    out = pl.pallas_call(
        _out_kernel,
        grid=(nlt,),
        in_specs=[
            pl.BlockSpec((1, 3), lambda i: (0, 0)),
            pl.BlockSpec((H, U), lambda i: (0, 0)),
            pl.BlockSpec((H, U, DH), lambda i: (0, 0, 0)),
            pl.BlockSpec((H, TQ, DH), lambda i: (0, i, 0)),
            wspec, bspec,
        ],
        out_specs=pl.BlockSpec((TQ, D), lambda i: (i, 0)),
        out_shape=jax.ShapeDtypeStruct((L, D), f32),
    )(pm, top, sel, comb, Wo, b2(bo))

    return out.reshape(1, L, D), awg, awl


# speedup vs baseline: 2.4721x; 1.1449x over previous
import functools
import math

import jax
import jax.numpy as jnp
from jax import lax
from jax.experimental import pallas as pl

L = 2048
D = 768
H = 12
DH = 64
LC = 512
CR = 4
HALF = 64
U = 8
TQ = 256
SCALE = 1.0 / math.sqrt(DH)
LN_L = math.log(L)


def _dotT(a, b):
    return lax.dot_general(a, b, (((1,), (1,)), ((), ())))


def _dot(a, b):
    return lax.dot_general(a, b, (((1,), (0,)), ((), ())))


def _split_heads(x):
    return x.reshape(x.shape[0], H, DH).transpose(1, 0, 2)


def _proj_kernel(xq_ref, xk_ref, xv_ref, wq_ref, bq_ref, wk_ref, bk_ref,
                 wv_ref, bv_ref, wc1_ref, bc1_ref, wc2_ref, bc2_ref,
                 q_ref, k_ref, v_ref, kc_ref, vc_ref):
    q_t = _dotT(xq_ref[...], wq_ref[...]) + bq_ref[...]
    k_t = _dotT(xk_ref[...], wk_ref[...]) + bk_ref[...]
    v_t = _dotT(xv_ref[...], wv_ref[...]) + bv_ref[...]
    q_ref[...] = _split_heads(q_t)
    k_ref[...] = _split_heads(k_t)
    v_ref[...] = _split_heads(v_t)
    rows = lax.broadcasted_iota(jnp.int32, (TQ // CR, TQ), 0)
    cols = lax.broadcasted_iota(jnp.int32, (TQ // CR, TQ), 1)
    pool = jnp.where((cols >= rows * CR) & (cols < rows * CR + CR),
                     1.0 / CR, 0.0).astype(jnp.float32)
    k_c = _dot(pool, k_t)
    vc_ref[...] = _split_heads(_dot(pool, v_t))
    h1 = _dotT(k_c, wc1_ref[...]) + bc1_ref[...]
    g = 0.5 * h1 * (1.0 + lax.erf(h1 / math.sqrt(2.0)))
    kc_ref[...] = _split_heads(_dotT(g, wc2_ref[...]) + bc2_ref[...])


def _attn_kernel(pm_ref, q_ref, k_ref, v_ref, kg_ref, vg_ref,
                 awg_ref, awl_ref, comb_ref, imp_ref):
    qi = pl.program_id(1)
    pm = pm_ref[...]
    e = jnp.exp(pm - jnp.max(pm))
    pw = e / jnp.sum(e)
    pw0 = pw[0, 0]
    pw1 = pw[0, 1]

    q = q_ref[0]

    sg = _dotT(q, kg_ref[0]) * SCALE
    pg = jnp.exp(sg - jnp.max(sg, axis=1, keepdims=True))
    awg = pg / jnp.sum(pg, axis=1, keepdims=True)
    awg_ref[0, 0] = awg
    g_out = _dot(awg, vg_ref[0])

    s = _dotT(q, k_ref[0]) * SCALE
    ms = jnp.max(s, axis=1, keepdims=True)
    p = jnp.exp(s - ms)
    sum_p = jnp.sum(p, axis=1, keepdims=True)
    lse = jnp.log(sum_p) + ms
    mean_s = jnp.sum(s, axis=1, keepdims=True) * (1.0 / L)
    imp = (lse - LN_L) - mean_s
    imp_ref[0, 0, pl.ds(qi * TQ, TQ)] = imp[:, 0]

    BW = TQ + 2 * HALF
    start = jnp.clip(qi * TQ - HALF, 0, L - BW)
    rows = qi * TQ + lax.broadcasted_iota(jnp.int32, (TQ, BW), 0)
    cols = start + lax.broadcasted_iota(jnp.int32, (TQ, BW), 1)
    band = jnp.abs(rows - cols) <= HALF
    k_win = k_ref[0, pl.ds(start, BW), :]
    p_win = jnp.exp(_dotT(q, k_win) * SCALE - ms)
    pb = jnp.where(band, p_win, 0.0)
    inv_denom = 1.0 / jnp.sum(pb, axis=1, keepdims=True)
    awl_win = pb * inv_denom
    rows_f = lax.broadcasted_iota(jnp.int32, (TQ, L), 0) + qi * TQ
    cols_f = lax.broadcasted_iota(jnp.int32, (TQ, L), 1)
    band_f = jnp.abs(rows_f - cols_f) <= HALF
    awl_ref[0, 0] = jnp.where(band_f, p, 0.0) * inv_denom
    v_win = v_ref[0, pl.ds(start, BW), :]
    l_out = _dot(awl_win, v_win)

    comb_ref[0] = pw0 * g_out + pw1 * l_out


def _topk_kernel(imp_ref, top_ref):
    x = imp_ref[:, 0, :]
    idx = lax.broadcasted_iota(jnp.int32, (H, L), 1)
    cols = []
    for _ in range(U):
        m = jnp.max(x, axis=1, keepdims=True)
        cand = jnp.where(x == m, idx, L)
        am = jnp.min(cand, axis=1)
        cols.append(am.reshape(H, 1))
        x = jnp.where(idx == am[:, None], -jnp.inf, x)
    top_ref[...] = jnp.concatenate(cols, axis=1)


def _sel_kernel(top_ref, q_ref, k_ref, v_ref, sel_ref):
    h = pl.program_id(0)
    t = top_ref[pl.ds(h, 1), :]
    colid = lax.broadcasted_iota(jnp.int32, (L, U), 0)
    onehot = (colid == t).astype(jnp.float32)
    q_sel = lax.dot_general(onehot, q_ref[0],
                            (((0,), (0,)), ((), ())))
    s = _dotT(q_sel, k_ref[0]) * SCALE
    p = jnp.exp(s - jnp.max(s, axis=1, keepdims=True))
    aw = p / jnp.sum(p, axis=1, keepdims=True)
    sel_ref[0] = _dot(aw, v_ref[0])


def _out_kernel(pm_ref, top_ref, sel_ref, comb_ref, wo_ref, bo_ref, out_ref):
    li = pl.program_id(0)
    pm = pm_ref[...]
    e = jnp.exp(pm - jnp.max(pm))
    pw = e / jnp.sum(e)
    pw2 = pw[0, 2]
    rows = li * TQ + lax.broadcasted_iota(jnp.int32, (TQ, U), 0)
    adds = []
    for h in range(H):
        th = top_ref[h:h + 1, :]
        oh = (rows == th).astype(jnp.float32)
        adds.append(_dot(oh, sel_ref[h]))
    addm = jnp.concatenate(adds, axis=1)
    comb = comb_ref[...]
    x = comb.transpose(1, 0, 2).reshape(TQ, D) + pw2 * addm
    out_ref[...] = _dotT(x, wo_ref[...]) + bo_ref[...]


def kernel(query, key, value, Wq, bq, Wk, bk, Wv, bv, Wo, bo,
           Wc1, bc1, Wc2, bc2, path_mixer):
    f32 = jnp.float32
    xq = query.reshape(L, D)
    xk = key.reshape(L, D)
    xv = value.reshape(L, D)
    b2 = lambda b: b.reshape(1, D)
    pm = path_mixer.reshape(1, 3)

    wspec = pl.BlockSpec((D, D), lambda *_: (0, 0))
    bspec = pl.BlockSpec((1, D), lambda *_: (0, 0))
    nlt = L // TQ

    q, k, v, kc, vc = pl.pallas_call(
        _proj_kernel,
        grid=(nlt,),
        in_specs=[
            pl.BlockSpec((TQ, D), lambda i: (i, 0)),
            pl.BlockSpec((TQ, D), lambda i: (i, 0)),
            pl.BlockSpec((TQ, D), lambda i: (i, 0)),
            wspec, bspec, wspec, bspec, wspec, bspec,
            wspec, bspec, wspec, bspec,
        ],
        out_specs=[
            pl.BlockSpec((H, TQ, DH), lambda i: (0, i, 0)),
            pl.BlockSpec((H, TQ, DH), lambda i: (0, i, 0)),
            pl.BlockSpec((H, TQ, DH), lambda i: (0, i, 0)),
            pl.BlockSpec((H, TQ // CR, DH), lambda i: (0, i, 0)),
            pl.BlockSpec((H, TQ // CR, DH), lambda i: (0, i, 0)),
        ],
        out_shape=[
            jax.ShapeDtypeStruct((H, L, DH), f32),
            jax.ShapeDtypeStruct((H, L, DH), f32),
            jax.ShapeDtypeStruct((H, L, DH), f32),
            jax.ShapeDtypeStruct((H, LC, DH), f32),
            jax.ShapeDtypeStruct((H, LC, DH), f32),
        ],
    )(xq, xk, xv, Wq, b2(bq), Wk, b2(bk), Wv, b2(bv),
      Wc1, b2(bc1), Wc2, b2(bc2))

    awg, awl, comb, imp = pl.pallas_call(
        _attn_kernel,
        grid=(H, nlt),
        in_specs=[
            pl.BlockSpec((1, 3), lambda h, i: (0, 0)),
            pl.BlockSpec((1, TQ, DH), lambda h, i: (h, i, 0)),
            pl.BlockSpec((1, L, DH), lambda h, i: (h, 0, 0)),
            pl.BlockSpec((1, L, DH), lambda h, i: (h, 0, 0)),
            pl.BlockSpec((1, LC, DH), lambda h, i: (h, 0, 0)),
            pl.BlockSpec((1, LC, DH), lambda h, i: (h, 0, 0)),
        ],
        out_specs=[
            pl.BlockSpec((1, 1, TQ, LC), lambda h, i: (0, h, i, 0)),
            pl.BlockSpec((1, 1, TQ, L), lambda h, i: (0, h, i, 0)),
            pl.BlockSpec((1, TQ, DH), lambda h, i: (h, i, 0)),
            pl.BlockSpec((1, 1, L), lambda h, i: (h, 0, 0)),
        ],
        out_shape=[
            jax.ShapeDtypeStruct((1, H, L, LC), f32),
            jax.ShapeDtypeStruct((1, H, L, L), f32),
            jax.ShapeDtypeStruct((H, L, DH), f32),
            jax.ShapeDtypeStruct((H, 1, L), f32),
        ],
    )(pm, q, k, v, kc, vc)

    top = pl.pallas_call(
        _topk_kernel,
        grid=(1,),
        in_specs=[pl.BlockSpec((H, 1, L), lambda i: (0, 0, 0))],
        out_specs=pl.BlockSpec((H, U), lambda i: (0, 0)),
        out_shape=jax.ShapeDtypeStruct((H, U), jnp.int32),
    )(imp)

    sel = pl.pallas_call(
        _sel_kernel,
        grid=(H,),
        in_specs=[
            pl.BlockSpec((H, U), lambda h: (0, 0)),
            pl.BlockSpec((1, L, DH), lambda h: (h, 0, 0)),
            pl.BlockSpec((1, L, DH), lambda h: (h, 0, 0)),
            pl.BlockSpec((1, L, DH), lambda h: (h, 0, 0)),
        ],
        out_specs=pl.BlockSpec((1, U, DH), lambda h: (h, 0, 0)),
        out_shape=jax.ShapeDtypeStruct((H, U, DH), f32),
    )(top, q, k, v)

    out = pl.pallas_call(
        _out_kernel,
        grid=(nlt,),
        in_specs=[
            pl.BlockSpec((1, 3), lambda i: (0, 0)),
            pl.BlockSpec((H, U), lambda i: (0, 0)),
            pl.BlockSpec((H, U, DH), lambda i: (0, 0, 0)),
            pl.BlockSpec((H, TQ, DH), lambda i: (0, i, 0)),
            wspec, bspec,
        ],
        out_specs=pl.BlockSpec((TQ, D), lambda i: (i, 0)),
        out_shape=jax.ShapeDtypeStruct((L, D), f32),
    )(pm, top, sel, comb, Wo, b2(bo))

    return out.reshape(1, L, D), awg, awl


# speedup vs baseline: 2.8633x; 1.1582x over previous
import functools
import math

import jax
import jax.numpy as jnp
from jax import lax
from jax.experimental import pallas as pl

L = 2048
D = 768
H = 12
DH = 64
LC = 512
CR = 4
HALF = 64
U = 8
TQ = 256
BW = 512
SCALE = 1.0 / math.sqrt(DH)
LN_L = math.log(L)


def _dot(a, b):
    return lax.dot_general(a, b, (((1,), (0,)), ((), ())))


def _dotT(a, b):
    return lax.dot_general(a, b, (((1,), (1,)), ((), ())))


def _dot00(a, b):
    return lax.dot_general(a, b, (((0,), (0,)), ((), ())))


TP = 512


def _proj_kernel(xq_ref, xk_ref, xv_ref, wq_ref, bq_ref, wk_ref, bk_ref,
                 wv_ref, bv_ref, q_ref, k_ref, v_ref, kp_ref, vc_ref):
    xq = xq_ref[...]
    xk = xk_ref[...]
    xv = xv_ref[...]
    q_t = _dotT(wq_ref[...], xq) + bq_ref[...]
    k_t = _dotT(wk_ref[...], xk) + bk_ref[...]
    v_t = _dotT(wv_ref[...], xv) + bv_ref[...]
    q_ref[...] = q_t.reshape(H, DH, TP)
    k_ref[...] = k_t.reshape(H, DH, TP)
    v_ref[...] = v_t.reshape(H, DH, TP)
    rows = lax.broadcasted_iota(jnp.int32, (TP // CR, TP), 0)
    cols = lax.broadcasted_iota(jnp.int32, (TP // CR, TP), 1)
    pool = jnp.where((cols >= rows * CR) & (cols < rows * CR + CR),
                     1.0 / CR, 0.0).astype(jnp.float32)
    kp_ref[...] = _dotT(k_t, pool).reshape(H, DH, TP // CR)
    vc_ref[...] = _dotT(v_t, pool).reshape(H, DH, TP // CR)


def _mlp_kernel(kp_ref, wc1_ref, bc1_ref, wc2_ref, bc2_ref, kc_ref):
    tc = kp_ref.shape[2]
    k_c = kp_ref[...].reshape(D, tc)
    h1 = _dot(wc1_ref[...], k_c) + bc1_ref[...]
    g = 0.5 * h1 * (1.0 + lax.erf(h1 / math.sqrt(2.0)))
    kc_ref[...] = (_dot(wc2_ref[...], g) + bc2_ref[...]).reshape(H, DH, tc)


def _attn_kernel(pm_ref, q_ref, k_ref, v_ref, kg_ref, vg_ref,
                 awg_ref, awl_ref, comb_ref, imp_ref):
    qi = pl.program_id(1)
    pm = pm_ref[...]
    e = jnp.exp(pm - jnp.max(pm))
    pw = e / jnp.sum(e)
    pw0 = pw[0, 0]
    pw1 = pw[0, 1]

    q = q_ref[0]

    sg = _dot00(q, kg_ref[0]) * SCALE
    pg = jnp.exp(sg - jnp.max(sg, axis=1, keepdims=True))
    awg = pg / jnp.sum(pg, axis=1, keepdims=True)
    awg_ref[0, 0] = awg
    g_out = _dotT(vg_ref[0], awg)

    s = _dot00(q, k_ref[0]) * SCALE
    ms = jnp.max(s, axis=1, keepdims=True)
    p = jnp.exp(s - ms)
    sum_p = jnp.sum(p, axis=1, keepdims=True)
    lse = jnp.log(sum_p) + ms
    mean_s = jnp.sum(s, axis=1, keepdims=True) * (1.0 / L)
    imp = (lse - LN_L) - mean_s
    imp_ref[0, 0, pl.ds(qi * TQ, TQ)] = imp[:, 0]

    start = (2 * HALF) * jnp.clip(2 * qi - 1, 0, (L - BW) // (2 * HALF))
    rows = qi * TQ + lax.broadcasted_iota(jnp.int32, (TQ, BW), 0)
    cols = start + lax.broadcasted_iota(jnp.int32, (TQ, BW), 1)
    band = jnp.abs(rows - cols) <= HALF
    k_win = k_ref[0, :, pl.ds(start, BW)]
    p_win = jnp.exp(_dot00(q, k_win) * SCALE - ms)
    pb = jnp.where(band, p_win, 0.0)
    inv_denom = 1.0 / jnp.sum(pb, axis=1, keepdims=True)
    awl_win = pb * inv_denom
    awl_ref[0, 0] = jnp.zeros((TQ, L), jnp.float32)
    awl_ref[0, 0, :, pl.ds(start, BW)] = awl_win
    v_win = v_ref[0, :, pl.ds(start, BW)]
    l_out = _dotT(v_win, awl_win)

    comb_ref[0] = pw0 * g_out + pw1 * l_out


def _topk_kernel(imp_ref, top_ref, topt_ref):
    x = imp_ref[:, 0, :]
    idx = lax.broadcasted_iota(jnp.int32, (H, L), 1)
    cols = []
    for r in range(U):
        m = jnp.max(x, axis=1, keepdims=True)
        cand = jnp.where(x == m, idx, L)
        am = jnp.min(cand, axis=1)
        cols.append(am.reshape(H, 1))
        topt_ref[r:r + 1, :] = am.reshape(1, H)
        x = jnp.where(idx == am[:, None], -jnp.inf, x)
    top_ref[...] = jnp.concatenate(cols, axis=1)


def _sel_kernel(top_ref, q_ref, k_ref, v_ref, sel_ref):
    h = pl.program_id(0)
    t = top_ref[pl.ds(h, 1), :]
    colid = lax.broadcasted_iota(jnp.int32, (L, U), 0)
    onehot = (colid == t).astype(jnp.float32)
    q_sel = _dot(q_ref[0], onehot)
    s = _dot00(q_sel, k_ref[0]) * SCALE
    p = jnp.exp(s - jnp.max(s, axis=1, keepdims=True))
    aw = p / jnp.sum(p, axis=1, keepdims=True)
    sel_ref[0] = _dotT(v_ref[0], aw)


def _out_kernel(pm_ref, topt_ref, sel_ref, comb_ref, wo_ref, bo_ref, out_ref):
    li = pl.program_id(0)
    pm = pm_ref[...]
    e = jnp.exp(pm - jnp.max(pm))
    pw = e / jnp.sum(e)
    pw2 = pw[0, 2]
    rows = li * TQ + lax.broadcasted_iota(jnp.int32, (U, TQ), 1)
    topt = topt_ref[...]
    parts = []
    for h in range(H):
        oh = (rows == topt[:, h:h + 1]).astype(jnp.float32)
        parts.append(_dot(sel_ref[h], oh))
    sadd = jnp.concatenate(parts, axis=0)
    x_t = comb_ref[...].reshape(D, TQ) + pw2 * sadd
    out = lax.dot_general(x_t, wo_ref[...], (((0,), (1,)), ((), ())))
    out_ref[...] = out + bo_ref[...]


def kernel(query, key, value, Wq, bq, Wk, bk, Wv, bv, Wo, bo,
           Wc1, bc1, Wc2, bc2, path_mixer):
    f32 = jnp.float32
    xq = query.reshape(L, D)
    xk = key.reshape(L, D)
    xv = value.reshape(L, D)
    b2 = lambda b: b.reshape(1, D)
    bcol = lambda b: b.reshape(D, 1)
    pm = path_mixer.reshape(1, 3)

    wspec = pl.BlockSpec((D, D), lambda *_: (0, 0))
    bspec = pl.BlockSpec((1, D), lambda *_: (0, 0))
    bcspec = pl.BlockSpec((D, 1), lambda *_: (0, 0))
    nlt = L // TQ

    q, k, v, kp, vc = pl.pallas_call(
        _proj_kernel,
        grid=(L // TP,),
        in_specs=[
            pl.BlockSpec((TP, D), lambda i: (i, 0)),
            pl.BlockSpec((TP, D), lambda i: (i, 0)),
            pl.BlockSpec((TP, D), lambda i: (i, 0)),
            wspec, bcspec, wspec, bcspec, wspec, bcspec,
        ],
        out_specs=[
            pl.BlockSpec((H, DH, TP), lambda i: (0, 0, i)),
            pl.BlockSpec((H, DH, TP), lambda i: (0, 0, i)),
            pl.BlockSpec((H, DH, TP), lambda i: (0, 0, i)),
            pl.BlockSpec((H, DH, TP // CR), lambda i: (0, 0, i)),
            pl.BlockSpec((H, DH, TP // CR), lambda i: (0, 0, i)),
        ],
        out_shape=[
            jax.ShapeDtypeStruct((H, DH, L), f32),
            jax.ShapeDtypeStruct((H, DH, L), f32),
            jax.ShapeDtypeStruct((H, DH, L), f32),
            jax.ShapeDtypeStruct((H, DH, LC), f32),
            jax.ShapeDtypeStruct((H, DH, LC), f32),
        ],
    )(xq, xk, xv, Wq, bcol(bq), Wk, bcol(bk), Wv, bcol(bv))

    TC = 128
    kc = pl.pallas_call(
        _mlp_kernel,
        grid=(LC // TC,),
        in_specs=[
            pl.BlockSpec((H, DH, TC), lambda i: (0, 0, i)),
            wspec, bcspec, wspec, bcspec,
        ],
        out_specs=pl.BlockSpec((H, DH, TC), lambda i: (0, 0, i)),
        out_shape=jax.ShapeDtypeStruct((H, DH, LC), f32),
    )(kp, Wc1, bcol(bc1), Wc2, bcol(bc2))

    awg, awl, comb, imp = pl.pallas_call(
        _attn_kernel,
        grid=(H, nlt),
        in_specs=[
            pl.BlockSpec((1, 3), lambda h, i: (0, 0)),
            pl.BlockSpec((1, DH, TQ), lambda h, i: (h, 0, i)),
            pl.BlockSpec((1, DH, L), lambda h, i: (h, 0, 0)),
            pl.BlockSpec((1, DH, L), lambda h, i: (h, 0, 0)),
            pl.BlockSpec((1, DH, LC), lambda h, i: (h, 0, 0)),
            pl.BlockSpec((1, DH, LC), lambda h, i: (h, 0, 0)),
        ],
        out_specs=[
            pl.BlockSpec((1, 1, TQ, LC), lambda h, i: (0, h, i, 0)),
            pl.BlockSpec((1, 1, TQ, L), lambda h, i: (0, h, i, 0)),
            pl.BlockSpec((1, DH, TQ), lambda h, i: (h, 0, i)),
            pl.BlockSpec((1, 1, L), lambda h, i: (h, 0, 0)),
        ],
        out_shape=[
            jax.ShapeDtypeStruct((1, H, L, LC), f32),
            jax.ShapeDtypeStruct((1, H, L, L), f32),
            jax.ShapeDtypeStruct((H, DH, L), f32),
            jax.ShapeDtypeStruct((H, 1, L), f32),
        ],
    )(pm, q, k, v, kc, vc)

    top, topt = pl.pallas_call(
        _topk_kernel,
        grid=(1,),
        in_specs=[pl.BlockSpec((H, 1, L), lambda i: (0, 0, 0))],
        out_specs=[
            pl.BlockSpec((H, U), lambda i: (0, 0)),
            pl.BlockSpec((U, H), lambda i: (0, 0)),
        ],
        out_shape=[
            jax.ShapeDtypeStruct((H, U), jnp.int32),
            jax.ShapeDtypeStruct((U, H), jnp.int32),
        ],
    )(imp)

    sel = pl.pallas_call(
        _sel_kernel,
        grid=(H,),
        in_specs=[
            pl.BlockSpec((H, U), lambda h: (0, 0)),
            pl.BlockSpec((1, DH, L), lambda h: (h, 0, 0)),
            pl.BlockSpec((1, DH, L), lambda h: (h, 0, 0)),
            pl.BlockSpec((1, DH, L), lambda h: (h, 0, 0)),
        ],
        out_specs=pl.BlockSpec((1, DH, U), lambda h: (h, 0, 0)),
        out_shape=jax.ShapeDtypeStruct((H, DH, U), f32),
    )(top, q, k, v)

    out = pl.pallas_call(
        _out_kernel,
        grid=(nlt,),
        in_specs=[
            pl.BlockSpec((1, 3), lambda i: (0, 0)),
            pl.BlockSpec((U, H), lambda i: (0, 0)),
            pl.BlockSpec((H, DH, U), lambda i: (0, 0, 0)),
            pl.BlockSpec((H, DH, TQ), lambda i: (0, 0, i)),
            wspec, bspec,
        ],
        out_specs=pl.BlockSpec((TQ, D), lambda i: (i, 0)),
        out_shape=jax.ShapeDtypeStruct((L, D), f32),
    )(pm, topt, sel, comb, Wo, b2(bo))

    return out.reshape(1, L, D), awg, awl


# speedup vs baseline: 2.9514x; 1.0308x over previous
import functools
import math

import jax
import jax.numpy as jnp
from jax import lax
from jax.experimental import pallas as pl

L = 2048
D = 768
H = 12
DH = 64
LC = 512
CR = 4
HALF = 64
U = 8
TQ = 256
BW = 512
SCALE = 1.0 / math.sqrt(DH)
LN_L = math.log(L)


def _dot(a, b):
    return lax.dot_general(a, b, (((1,), (0,)), ((), ())))


def _dotT(a, b):
    return lax.dot_general(a, b, (((1,), (1,)), ((), ())))


def _dot00(a, b):
    return lax.dot_general(a, b, (((0,), (0,)), ((), ())))


TP = 512


def _proj_kernel(xq_ref, xk_ref, xv_ref, wq_ref, bq_ref, wk_ref, bk_ref,
                 wv_ref, bv_ref, q_ref, k_ref, v_ref, kp_ref, vc_ref):
    xq = xq_ref[...]
    xk = xk_ref[...]
    xv = xv_ref[...]
    q_t = _dotT(wq_ref[...], xq) + bq_ref[...]
    k_t = _dotT(wk_ref[...], xk) + bk_ref[...]
    v_t = _dotT(wv_ref[...], xv) + bv_ref[...]
    q_ref[...] = q_t.reshape(H, DH, TP)
    k_ref[...] = k_t.reshape(H, DH, TP)
    v_ref[...] = v_t.reshape(H, DH, TP)
    rows = lax.broadcasted_iota(jnp.int32, (TP // CR, TP), 0)
    cols = lax.broadcasted_iota(jnp.int32, (TP // CR, TP), 1)
    pool = jnp.where((cols >= rows * CR) & (cols < rows * CR + CR),
                     1.0 / CR, 0.0).astype(jnp.float32)
    kp_ref[...] = _dotT(k_t, pool).reshape(H, DH, TP // CR)
    vc_ref[...] = _dotT(v_t, pool).reshape(H, DH, TP // CR)


def _mlp_kernel(kp_ref, wc1_ref, bc1_ref, wc2_ref, bc2_ref, kc_ref):
    tc = kp_ref.shape[2]
    k_c = kp_ref[...].reshape(D, tc)
    h1 = _dot(wc1_ref[...], k_c) + bc1_ref[...]
    g = 0.5 * h1 * (1.0 + lax.erf(h1 / math.sqrt(2.0)))
    kc_ref[...] = (_dot(wc2_ref[...], g) + bc2_ref[...]).reshape(H, DH, tc)


def _attn_kernel(pm_ref, q_ref, k_ref, v_ref, kg_ref, vg_ref,
                 awg_ref, awl_ref, comb_ref, imp_ref):
    qi = pl.program_id(1)
    pm = pm_ref[...]
    e = jnp.exp(pm - jnp.max(pm))
    pw = e / jnp.sum(e)
    pw0 = pw[0, 0]
    pw1 = pw[0, 1]

    q = q_ref[0] * SCALE

    sg = _dot00(q, kg_ref[0])
    pg = jnp.exp(sg - jnp.max(sg, axis=1, keepdims=True))
    awg = pg * (1.0 / jnp.sum(pg, axis=1, keepdims=True))
    awg_ref[0, 0] = awg
    g_out = _dotT(vg_ref[0], awg)

    s = _dot00(q, k_ref[0])
    ms = jnp.max(s, axis=1, keepdims=True)
    p = jnp.exp(s - ms)
    sum_p = jnp.sum(p, axis=1, keepdims=True)
    lse = jnp.log(sum_p) + ms
    mean_s = jnp.sum(s, axis=1, keepdims=True) * (1.0 / L)
    imp = (lse - LN_L) - mean_s
    imp_ref[0, 0, pl.ds(qi * TQ, TQ)] = imp[:, 0]

    start = (2 * HALF) * jnp.clip(2 * qi - 1, 0, (L - BW) // (2 * HALF))
    rows = qi * TQ + lax.broadcasted_iota(jnp.int32, (TQ, BW), 0)
    cols = start + lax.broadcasted_iota(jnp.int32, (TQ, BW), 1)
    band = jnp.abs(rows - cols) <= HALF
    k_win = k_ref[0, :, pl.ds(start, BW)]
    p_win = jnp.exp(_dot00(q, k_win) - ms)
    pb = jnp.where(band, p_win, 0.0)
    inv_denom = 1.0 / jnp.sum(pb, axis=1, keepdims=True)
    awl_win = pb * inv_denom
    awl_ref[0, 0] = jnp.zeros((TQ, L), jnp.float32)
    awl_ref[0, 0, :, pl.ds(start, BW)] = awl_win
    v_win = v_ref[0, :, pl.ds(start, BW)]
    l_out = _dotT(v_win, awl_win)

    comb_ref[0] = pw0 * g_out + pw1 * l_out


def _topk_kernel(imp_ref, top_ref, topt_ref):
    x = imp_ref[:, 0, :]
    idx = lax.broadcasted_iota(jnp.int32, (H, L), 1)
    cols = []
    for r in range(U):
        m = jnp.max(x, axis=1, keepdims=True)
        cand = jnp.where(x == m, idx, L)
        am = jnp.min(cand, axis=1)
        cols.append(am.reshape(H, 1))
        topt_ref[r:r + 1, :] = am.reshape(1, H)
        x = jnp.where(idx == am[:, None], -jnp.inf, x)
    top_ref[...] = jnp.concatenate(cols, axis=1)


def _sel_kernel(top_ref, q_ref, k_ref, v_ref, sel_ref):
    h = pl.program_id(0)
    t = top_ref[pl.ds(h, 1), :]
    colid = lax.broadcasted_iota(jnp.int32, (L, U), 0)
    onehot = (colid == t).astype(jnp.float32)
    q_sel = _dot(q_ref[0], onehot)
    s = _dot00(q_sel, k_ref[0]) * SCALE
    p = jnp.exp(s - jnp.max(s, axis=1, keepdims=True))
    aw = p / jnp.sum(p, axis=1, keepdims=True)
    sel_ref[0] = _dotT(v_ref[0], aw)


def _out_kernel(pm_ref, topt_ref, sel_ref, comb_ref, wo_ref, bo_ref, out_ref):
    li = pl.program_id(0)
    pm = pm_ref[...]
    e = jnp.exp(pm - jnp.max(pm))
    pw = e / jnp.sum(e)
    pw2 = pw[0, 2]
    rows = li * TQ + lax.broadcasted_iota(jnp.int32, (U, TQ), 1)
    topt = topt_ref[...]
    parts = []
    for h in range(H):
        oh = (rows == topt[:, h:h + 1]).astype(jnp.float32)
        parts.append(_dot(sel_ref[h], oh))
    sadd = jnp.concatenate(parts, axis=0)
    x_t = comb_ref[...].reshape(D, TQ) + pw2 * sadd
    out = lax.dot_general(x_t, wo_ref[...], (((0,), (1,)), ((), ())))
    out_ref[...] = out + bo_ref[...]


def kernel(query, key, value, Wq, bq, Wk, bk, Wv, bv, Wo, bo,
           Wc1, bc1, Wc2, bc2, path_mixer):
    f32 = jnp.float32
    xq = query.reshape(L, D)
    xk = key.reshape(L, D)
    xv = value.reshape(L, D)
    b2 = lambda b: b.reshape(1, D)
    bcol = lambda b: b.reshape(D, 1)
    pm = path_mixer.reshape(1, 3)

    wspec = pl.BlockSpec((D, D), lambda *_: (0, 0))
    bspec = pl.BlockSpec((1, D), lambda *_: (0, 0))
    bcspec = pl.BlockSpec((D, 1), lambda *_: (0, 0))
    nlt = L // TQ

    q, k, v, kp, vc = pl.pallas_call(
        _proj_kernel,
        grid=(L // TP,),
        in_specs=[
            pl.BlockSpec((TP, D), lambda i: (i, 0)),
            pl.BlockSpec((TP, D), lambda i: (i, 0)),
            pl.BlockSpec((TP, D), lambda i: (i, 0)),
            wspec, bcspec, wspec, bcspec, wspec, bcspec,
        ],
        out_specs=[
            pl.BlockSpec((H, DH, TP), lambda i: (0, 0, i)),
            pl.BlockSpec((H, DH, TP), lambda i: (0, 0, i)),
            pl.BlockSpec((H, DH, TP), lambda i: (0, 0, i)),
            pl.BlockSpec((H, DH, TP // CR), lambda i: (0, 0, i)),
            pl.BlockSpec((H, DH, TP // CR), lambda i: (0, 0, i)),
        ],
        out_shape=[
            jax.ShapeDtypeStruct((H, DH, L), f32),
            jax.ShapeDtypeStruct((H, DH, L), f32),
            jax.ShapeDtypeStruct((H, DH, L), f32),
            jax.ShapeDtypeStruct((H, DH, LC), f32),
            jax.ShapeDtypeStruct((H, DH, LC), f32),
        ],
    )(xq, xk, xv, Wq, bcol(bq), Wk, bcol(bk), Wv, bcol(bv))

    TC = 128
    kc = pl.pallas_call(
        _mlp_kernel,
        grid=(LC // TC,),
        in_specs=[
            pl.BlockSpec((H, DH, TC), lambda i: (0, 0, i)),
            wspec, bcspec, wspec, bcspec,
        ],
        out_specs=pl.BlockSpec((H, DH, TC), lambda i: (0, 0, i)),
        out_shape=jax.ShapeDtypeStruct((H, DH, LC), f32),
    )(kp, Wc1, bcol(bc1), Wc2, bcol(bc2))

    awg, awl, comb, imp = pl.pallas_call(
        _attn_kernel,
        grid=(H, nlt),
        in_specs=[
            pl.BlockSpec((1, 3), lambda h, i: (0, 0)),
            pl.BlockSpec((1, DH, TQ), lambda h, i: (h, 0, i)),
            pl.BlockSpec((1, DH, L), lambda h, i: (h, 0, 0)),
            pl.BlockSpec((1, DH, L), lambda h, i: (h, 0, 0)),
            pl.BlockSpec((1, DH, LC), lambda h, i: (h, 0, 0)),
            pl.BlockSpec((1, DH, LC), lambda h, i: (h, 0, 0)),
        ],
        out_specs=[
            pl.BlockSpec((1, 1, TQ, LC), lambda h, i: (0, h, i, 0)),
            pl.BlockSpec((1, 1, TQ, L), lambda h, i: (0, h, i, 0)),
            pl.BlockSpec((1, DH, TQ), lambda h, i: (h, 0, i)),
            pl.BlockSpec((1, 1, L), lambda h, i: (h, 0, 0)),
        ],
        out_shape=[
            jax.ShapeDtypeStruct((1, H, L, LC), f32),
            jax.ShapeDtypeStruct((1, H, L, L), f32),
            jax.ShapeDtypeStruct((H, DH, L), f32),
            jax.ShapeDtypeStruct((H, 1, L), f32),
        ],
    )(pm, q, k, v, kc, vc)

    top, topt = pl.pallas_call(
        _topk_kernel,
        grid=(1,),
        in_specs=[pl.BlockSpec((H, 1, L), lambda i: (0, 0, 0))],
        out_specs=[
            pl.BlockSpec((H, U), lambda i: (0, 0)),
            pl.BlockSpec((U, H), lambda i: (0, 0)),
        ],
        out_shape=[
            jax.ShapeDtypeStruct((H, U), jnp.int32),
            jax.ShapeDtypeStruct((U, H), jnp.int32),
        ],
    )(imp)

    sel = pl.pallas_call(
        _sel_kernel,
        grid=(H,),
        in_specs=[
            pl.BlockSpec((H, U), lambda h: (0, 0)),
            pl.BlockSpec((1, DH, L), lambda h: (h, 0, 0)),
            pl.BlockSpec((1, DH, L), lambda h: (h, 0, 0)),
            pl.BlockSpec((1, DH, L), lambda h: (h, 0, 0)),
        ],
        out_specs=pl.BlockSpec((1, DH, U), lambda h: (h, 0, 0)),
        out_shape=jax.ShapeDtypeStruct((H, DH, U), f32),
    )(top, q, k, v)

    out = pl.pallas_call(
        _out_kernel,
        grid=(nlt,),
        in_specs=[
            pl.BlockSpec((1, 3), lambda i: (0, 0)),
            pl.BlockSpec((U, H), lambda i: (0, 0)),
            pl.BlockSpec((H, DH, U), lambda i: (0, 0, 0)),
            pl.BlockSpec((H, DH, TQ), lambda i: (0, 0, i)),
            wspec, bspec,
        ],
        out_specs=pl.BlockSpec((TQ, D), lambda i: (i, 0)),
        out_shape=jax.ShapeDtypeStruct((L, D), f32),
    )(pm, topt, sel, comb, Wo, b2(bo))

    return out.reshape(1, L, D), awg, awl


# speedup vs baseline: 3.0897x; 1.0468x over previous
import functools
import math

import jax
import jax.numpy as jnp
from jax import lax
from jax.experimental import pallas as pl

L = 2048
D = 768
H = 12
DH = 64
LC = 512
CR = 4
HALF = 64
U = 8
TQ = 256
TA = 512
BW = 768
SCALE = 1.0 / math.sqrt(DH)
LN_L = math.log(L)


def _dot(a, b):
    return lax.dot_general(a, b, (((1,), (0,)), ((), ())))


def _dotT(a, b):
    return lax.dot_general(a, b, (((1,), (1,)), ((), ())))


def _dot00(a, b):
    return lax.dot_general(a, b, (((0,), (0,)), ((), ())))


TP = 512


def _proj_kernel(xq_ref, xk_ref, xv_ref, wq_ref, bq_ref, wk_ref, bk_ref,
                 wv_ref, bv_ref, q_ref, k_ref, v_ref, kp_ref, vc_ref):
    xq = xq_ref[...]
    xk = xk_ref[...]
    xv = xv_ref[...]
    q_t = _dotT(wq_ref[...], xq) + bq_ref[...]
    k_t = _dotT(wk_ref[...], xk) + bk_ref[...]
    v_t = _dotT(wv_ref[...], xv) + bv_ref[...]
    q_ref[...] = q_t.reshape(H, DH, TP)
    k_ref[...] = k_t.reshape(H, DH, TP)
    v_ref[...] = v_t.reshape(H, DH, TP)
    rows = lax.broadcasted_iota(jnp.int32, (TP // CR, TP), 0)
    cols = lax.broadcasted_iota(jnp.int32, (TP // CR, TP), 1)
    pool = jnp.where((cols >= rows * CR) & (cols < rows * CR + CR),
                     1.0 / CR, 0.0).astype(jnp.float32)
    kp_ref[...] = _dotT(k_t, pool).reshape(H, DH, TP // CR)
    vc_ref[...] = _dotT(v_t, pool).reshape(H, DH, TP // CR)


def _mlp_kernel(kp_ref, wc1_ref, bc1_ref, wc2_ref, bc2_ref, kc_ref):
    tc = kp_ref.shape[2]
    k_c = kp_ref[...].reshape(D, tc)
    h1 = _dot(wc1_ref[...], k_c) + bc1_ref[...]
    g = 0.5 * h1 * (1.0 + lax.erf(h1 / math.sqrt(2.0)))
    kc_ref[...] = (_dot(wc2_ref[...], g) + bc2_ref[...]).reshape(H, DH, tc)


def _attn_kernel(pm_ref, q_ref, k_ref, v_ref, kg_ref, vg_ref,
                 awg_ref, awl_ref, comb_ref, imp_ref):
    qi = pl.program_id(1)
    pm = pm_ref[...]
    e = jnp.exp(pm - jnp.max(pm))
    pw = e / jnp.sum(e)
    pw0 = pw[0, 0]
    pw1 = pw[0, 1]

    q = q_ref[0] * SCALE

    sg = _dot00(q, kg_ref[0])
    pg = jnp.exp(sg - jnp.max(sg, axis=1, keepdims=True))
    awg = pg * (1.0 / jnp.sum(pg, axis=1, keepdims=True))
    awg_ref[0, 0] = awg
    g_out = _dotT(vg_ref[0], awg)

    s = _dot00(q, k_ref[0])
    ms = jnp.max(s, axis=1, keepdims=True)
    p = jnp.exp(s - ms)
    sum_p = jnp.sum(p, axis=1, keepdims=True)
    lse = jnp.log(sum_p) + ms
    mean_s = jnp.sum(s, axis=1, keepdims=True) * (1.0 / L)
    imp = (lse - LN_L) - mean_s
    imp_ref[0, 0, pl.ds(qi * TA, TA)] = imp[:, 0]

    start = (2 * HALF) * jnp.clip(4 * qi - 1, 0, (L - BW) // (2 * HALF))
    rows = qi * TA + lax.broadcasted_iota(jnp.int32, (TA, BW), 0)
    cols = start + lax.broadcasted_iota(jnp.int32, (TA, BW), 1)
    band = jnp.abs(rows - cols) <= HALF
    k_win = k_ref[0, :, pl.ds(start, BW)]
    p_win = jnp.exp(_dot00(q, k_win) - ms)
    pb = jnp.where(band, p_win, 0.0)
    inv_denom = 1.0 / jnp.sum(pb, axis=1, keepdims=True)
    awl_win = pb * inv_denom
    awl_ref[0, 0] = jnp.zeros((TA, L), jnp.float32)
    awl_ref[0, 0, :, pl.ds(start, BW)] = awl_win
    v_win = v_ref[0, :, pl.ds(start, BW)]
    l_out = _dotT(v_win, awl_win)

    comb_ref[0] = pw0 * g_out + pw1 * l_out


def _topk_kernel(imp_ref, top_ref, topt_ref):
    x = imp_ref[:, 0, :]
    idx = lax.broadcasted_iota(jnp.int32, (H, L), 1)
    cols = []
    for r in range(U):
        m = jnp.max(x, axis=1, keepdims=True)
        cand = jnp.where(x == m, idx, L)
        am = jnp.min(cand, axis=1)
        cols.append(am.reshape(H, 1))
        topt_ref[r:r + 1, :] = am.reshape(1, H)
        x = jnp.where(idx == am[:, None], -jnp.inf, x)
    top_ref[...] = jnp.concatenate(cols, axis=1)


def _sel_kernel(top_ref, q_ref, k_ref, v_ref, sel_ref):
    h = pl.program_id(0)
    t = top_ref[pl.ds(h, 1), :]
    colid = lax.broadcasted_iota(jnp.int32, (L, U), 0)
    onehot = (colid == t).astype(jnp.float32)
    q_sel = _dot(q_ref[0], onehot)
    s = _dot00(q_sel, k_ref[0]) * SCALE
    p = jnp.exp(s - jnp.max(s, axis=1, keepdims=True))
    aw = p / jnp.sum(p, axis=1, keepdims=True)
    sel_ref[0] = _dotT(v_ref[0], aw)


def _out_kernel(pm_ref, topt_ref, sel_ref, comb_ref, wo_ref, bo_ref, out_ref):
    li = pl.program_id(0)
    pm = pm_ref[...]
    e = jnp.exp(pm - jnp.max(pm))
    pw = e / jnp.sum(e)
    pw2 = pw[0, 2]
    rows = li * TQ + lax.broadcasted_iota(jnp.int32, (U, TQ), 1)
    topt = topt_ref[...]
    parts = []
    for h in range(H):
        oh = (rows == topt[:, h:h + 1]).astype(jnp.float32)
        parts.append(_dot(sel_ref[h], oh))
    sadd = jnp.concatenate(parts, axis=0)
    x_t = comb_ref[...].reshape(D, TQ) + pw2 * sadd
    out = lax.dot_general(x_t, wo_ref[...], (((0,), (1,)), ((), ())))
    out_ref[...] = out + bo_ref[...]


def kernel(query, key, value, Wq, bq, Wk, bk, Wv, bv, Wo, bo,
           Wc1, bc1, Wc2, bc2, path_mixer):
    f32 = jnp.float32
    xq = query.reshape(L, D)
    xk = key.reshape(L, D)
    xv = value.reshape(L, D)
    b2 = lambda b: b.reshape(1, D)
    bcol = lambda b: b.reshape(D, 1)
    pm = path_mixer.reshape(1, 3)

    wspec = pl.BlockSpec((D, D), lambda *_: (0, 0))
    bspec = pl.BlockSpec((1, D), lambda *_: (0, 0))
    bcspec = pl.BlockSpec((D, 1), lambda *_: (0, 0))
    nlt = L // TQ

    q, k, v, kp, vc = pl.pallas_call(
        _proj_kernel,
        grid=(L // TP,),
        in_specs=[
            pl.BlockSpec((TP, D), lambda i: (i, 0)),
            pl.BlockSpec((TP, D), lambda i: (i, 0)),
            pl.BlockSpec((TP, D), lambda i: (i, 0)),
            wspec, bcspec, wspec, bcspec, wspec, bcspec,
        ],
        out_specs=[
            pl.BlockSpec((H, DH, TP), lambda i: (0, 0, i)),
            pl.BlockSpec((H, DH, TP), lambda i: (0, 0, i)),
            pl.BlockSpec((H, DH, TP), lambda i: (0, 0, i)),
            pl.BlockSpec((H, DH, TP // CR), lambda i: (0, 0, i)),
            pl.BlockSpec((H, DH, TP // CR), lambda i: (0, 0, i)),
        ],
        out_shape=[
            jax.ShapeDtypeStruct((H, DH, L), f32),
            jax.ShapeDtypeStruct((H, DH, L), f32),
            jax.ShapeDtypeStruct((H, DH, L), f32),
            jax.ShapeDtypeStruct((H, DH, LC), f32),
            jax.ShapeDtypeStruct((H, DH, LC), f32),
        ],
    )(xq, xk, xv, Wq, bcol(bq), Wk, bcol(bk), Wv, bcol(bv))

    TC = 128
    kc = pl.pallas_call(
        _mlp_kernel,
        grid=(LC // TC,),
        in_specs=[
            pl.BlockSpec((H, DH, TC), lambda i: (0, 0, i)),
            wspec, bcspec, wspec, bcspec,
        ],
        out_specs=pl.BlockSpec((H, DH, TC), lambda i: (0, 0, i)),
        out_shape=jax.ShapeDtypeStruct((H, DH, LC), f32),
    )(kp, Wc1, bcol(bc1), Wc2, bcol(bc2))

    nat = L // TA
    awg, awl, comb, imp = pl.pallas_call(
        _attn_kernel,
        grid=(H, nat),
        in_specs=[
            pl.BlockSpec((1, 3), lambda h, i: (0, 0)),
            pl.BlockSpec((1, DH, TA), lambda h, i: (h, 0, i)),
            pl.BlockSpec((1, DH, L), lambda h, i: (h, 0, 0)),
            pl.BlockSpec((1, DH, L), lambda h, i: (h, 0, 0)),
            pl.BlockSpec((1, DH, LC), lambda h, i: (h, 0, 0)),
            pl.BlockSpec((1, DH, LC), lambda h, i: (h, 0, 0)),
        ],
        out_specs=[
            pl.BlockSpec((1, 1, TA, LC), lambda h, i: (0, h, i, 0)),
            pl.BlockSpec((1, 1, TA, L), lambda h, i: (0, h, i, 0)),
            pl.BlockSpec((1, DH, TA), lambda h, i: (h, 0, i)),
            pl.BlockSpec((1, 1, L), lambda h, i: (h, 0, 0)),
        ],
        out_shape=[
            jax.ShapeDtypeStruct((1, H, L, LC), f32),
            jax.ShapeDtypeStruct((1, H, L, L), f32),
            jax.ShapeDtypeStruct((H, DH, L), f32),
            jax.ShapeDtypeStruct((H, 1, L), f32),
        ],
    )(pm, q, k, v, kc, vc)

    top, topt = pl.pallas_call(
        _topk_kernel,
        grid=(1,),
        in_specs=[pl.BlockSpec((H, 1, L), lambda i: (0, 0, 0))],
        out_specs=[
            pl.BlockSpec((H, U), lambda i: (0, 0)),
            pl.BlockSpec((U, H), lambda i: (0, 0)),
        ],
        out_shape=[
            jax.ShapeDtypeStruct((H, U), jnp.int32),
            jax.ShapeDtypeStruct((U, H), jnp.int32),
        ],
    )(imp)

    sel = pl.pallas_call(
        _sel_kernel,
        grid=(H,),
        in_specs=[
            pl.BlockSpec((H, U), lambda h: (0, 0)),
            pl.BlockSpec((1, DH, L), lambda h: (h, 0, 0)),
            pl.BlockSpec((1, DH, L), lambda h: (h, 0, 0)),
            pl.BlockSpec((1, DH, L), lambda h: (h, 0, 0)),
        ],
        out_specs=pl.BlockSpec((1, DH, U), lambda h: (h, 0, 0)),
        out_shape=jax.ShapeDtypeStruct((H, DH, U), f32),
    )(top, q, k, v)

    out = pl.pallas_call(
        _out_kernel,
        grid=(nlt,),
        in_specs=[
            pl.BlockSpec((1, 3), lambda i: (0, 0)),
            pl.BlockSpec((U, H), lambda i: (0, 0)),
            pl.BlockSpec((H, DH, U), lambda i: (0, 0, 0)),
            pl.BlockSpec((H, DH, TQ), lambda i: (0, 0, i)),
            wspec, bspec,
        ],
        out_specs=pl.BlockSpec((TQ, D), lambda i: (i, 0)),
        out_shape=jax.ShapeDtypeStruct((L, D), f32),
    )(pm, topt, sel, comb, Wo, b2(bo))

    return out.reshape(1, L, D), awg, awl
